# baseline (speedup 1.0000x reference)
import functools

import jax
import jax.numpy as jnp
from jax.experimental import pallas as pl
from jax.experimental.pallas import tpu as pltpu

B = 4
F = 128
S0 = 256
S1 = 256
H = 4
N = S0 + S1

_dotg = functools.partial(
    jax.lax.dot_general,
    precision=jax.lax.Precision.DEFAULT,
    preferred_element_type=jnp.float32,
)


def _dot(a, b):
    return _dotg(a, b, (((1,), (0,)), ((), ())))


def _lrelu(x):
    return jnp.maximum(x, 0.2 * x)


def _gat_body(x_ref,
              W0_ref, As0_ref, Ad0_ref, b0_ref, mW0_ref, mb0_ref,
              W1_ref, As1_ref, Ad1_ref, b1_ref, mW1_ref, mb1_ref,
              o_ref):
    x = x_ref[0]
    layers = (
        (W0_ref, As0_ref, Ad0_ref, b0_ref, mW0_ref, mb0_ref, False),
        (W1_ref, As1_ref, Ad1_ref, b1_ref, mW1_ref, mb1_ref, True),
    )
    for W_ref, As_ref, Ad_ref, bias_ref, mW_ref, mb_ref, cross in layers:
        h = _dot(x, W_ref[...])
        als = _dot(h, As_ref[...])
        ald = _dot(h, Ad_ref[...])
        msg_sets = []
        for s in (0, 1):
            dlo = s * S0
            slo = (1 - s) * S0 if cross else dlo
            als_src = als[slo:slo + S0, :]
            als_row = jnp.swapaxes(als_src, 0, 1)
            maxa = jnp.max(als_src, axis=0, keepdims=True)
            acc = jnp.zeros((S0, F), jnp.float32)
            for hi in range(H):
                hs = h[slo:slo + S0, hi * F:(hi + 1) * F]
                row = als_row[hi:hi + 1, :]
                col = ald[dlo:dlo + S0, hi:hi + 1]
                sc = _lrelu(row + col)
                m = _lrelu(maxa[0:1, hi:hi + 1] + col)
                if cross:
                    hd = h[dlo:dlo + S0, hi * F:(hi + 1) * F]
                    s_self = _lrelu(als[dlo:dlo + S0, hi:hi + 1] + col)
                    m = jnp.maximum(m, s_self)
                    e = jnp.exp(sc - m)
                    e_self = jnp.exp(s_self - m)
                    den = jnp.sum(e, axis=1, keepdims=True) + e_self + 1e-16
                    acc = acc + (_dot(e, hs) + e_self * hd) / den
                else:
                    e = jnp.exp(sc - m)
                    den = jnp.sum(e, axis=1, keepdims=True) + 1e-16
                    acc = acc + _dot(e, hs) / den
            msg_sets.append(acc)
        msg1 = jnp.concatenate(msg_sets, axis=0)
        msg1 = msg1 * (1.0 / H) + bias_ref[...]
        msg1 = jnp.maximum(msg1, 0.0)
        mW = mW_ref[...]
        msg2 = _dot(x, mW[:F, :]) + _dot(msg1, mW[F:, :]) + mb_ref[...]
        x = x + msg2
    o_ref[0] = x


@jax.jit
def kernel(desc0, desc1, W0, att_src0, att_dst0, b0, mlp_W0, mlp_b0,
           W1, att_src1, att_dst1, b1, mlp_W1, mlp_b1):
    x = jnp.concatenate([desc0, desc1], axis=2)
    xin = jnp.transpose(x, (0, 2, 1)).astype(jnp.float32)

    eye = jnp.eye(H, dtype=jnp.float32)
    bd = lambda a: jnp.einsum('kc,kj->kcj', a, eye).reshape(H * F, H)

    full = lambda a: pl.BlockSpec(a.shape, lambda b: (0,) * a.ndim)
    args = (W0, bd(att_src0), bd(att_dst0), b0.reshape(1, F), mlp_W0,
            mlp_b0.reshape(1, F),
            W1, bd(att_src1), bd(att_dst1), b1.reshape(1, F), mlp_W1,
            mlp_b1.reshape(1, F))

    out = pl.pallas_call(
        _gat_body,
        grid=(B,),
        in_specs=[pl.BlockSpec((1, N, F), lambda b: (b, 0, 0))]
        + [full(a) for a in args],
        out_specs=pl.BlockSpec((1, N, F), lambda b: (b, 0, 0)),
        out_shape=jax.ShapeDtypeStruct((B, N, F), jnp.float32),
        compiler_params=pltpu.CompilerParams(
            dimension_semantics=("parallel",)),
    )(xin, *args)

    xo = jnp.transpose(out, (0, 2, 1))
    return xo[:, :, :S0], xo[:, :, S0:]


# speedup vs baseline: 1.1551x; 1.1551x over previous
import functools

import jax
import jax.numpy as jnp
from jax.experimental import pallas as pl

B = 4
F = 128
S0 = 256
S1 = 256
H = 4
N = S0 + S1

_dotg = functools.partial(
    jax.lax.dot_general,
    precision=jax.lax.Precision.DEFAULT,
    preferred_element_type=jnp.float32,
)


def _dot(a, b):
    return _dotg(a, b, (((1,), (0,)), ((), ())))


def _dot_t(a, b):
    return _dotg(a, b, (((1,), (1,)), ((), ())))


def _lrelu(x):
    return jnp.maximum(x, 0.2 * x)


def _gat_body(x_ref,
              W0_ref, as0_ref, ad0_ref, b0_ref, mW0_ref, mb0_ref,
              W1_ref, as1_ref, ad1_ref, b1_ref, mW1_ref, mb1_ref,
              o_ref):
    x = x_ref[0]
    layers = (
        (W0_ref, as0_ref, ad0_ref, b0_ref, mW0_ref, mb0_ref, False),
        (W1_ref, as1_ref, ad1_ref, b1_ref, mW1_ref, mb1_ref, True),
    )
    for W_ref, as_ref, ad_ref, bias_ref, mW_ref, mb_ref, cross in layers:
        h = _dot(x, W_ref[...])
        msg_sets = []
        for s in (0, 1):
            dlo = s * S0
            slo = (1 - s) * S0 if cross else dlo
            acc = jnp.zeros((S0, F), jnp.float32)
            for hi in range(H):
                hs = h[slo:slo + S0, hi * F:(hi + 1) * F]
                hd = h[dlo:dlo + S0, hi * F:(hi + 1) * F]
                a_s = as_ref[hi:hi + 1, :]
                a_d = ad_ref[hi:hi + 1, :]
                row = _dot_t(a_s, hs)
                col = _dot_t(hd, a_d)
                sc = _lrelu(row + col)
                rmax = jnp.max(row, axis=1, keepdims=True)
                m = _lrelu(rmax + col)
                if cross:
                    s_self = _lrelu(_dot_t(hd, a_s) + col)
                    m = jnp.maximum(m, s_self)
                    e = jnp.exp(sc - m)
                    e_self = jnp.exp(s_self - m)
                    den = jnp.sum(e, axis=1, keepdims=True) + e_self + 1e-16
                    acc = acc + (_dot(e, hs) + e_self * hd) / den
                else:
                    e = jnp.exp(sc - m)
                    den = jnp.sum(e, axis=1, keepdims=True) + 1e-16
                    acc = acc + _dot(e, hs) / den
            msg_sets.append(acc)
        msg1 = jnp.concatenate(msg_sets, axis=0)
        msg1 = msg1 * (1.0 / H) + bias_ref[...]
        msg1 = jnp.maximum(msg1, 0.0)
        mW = mW_ref[...]
        msg2 = _dot(x, mW[:F, :]) + _dot(msg1, mW[F:, :]) + mb_ref[...]
        x = x + msg2
    o_ref[0] = x


@jax.jit
def kernel(desc0, desc1, W0, att_src0, att_dst0, b0, mlp_W0, mlp_b0,
           W1, att_src1, att_dst1, b1, mlp_W1, mlp_b1):
    x = jnp.concatenate([desc0, desc1], axis=2)
    xin = jnp.transpose(x, (0, 2, 1)).astype(jnp.float32)

    full = lambda a: pl.BlockSpec(a.shape, lambda b: (0,) * a.ndim)
    args = (W0, att_src0, att_dst0, b0.reshape(1, F), mlp_W0,
            mlp_b0.reshape(1, F),
            W1, att_src1, att_dst1, b1.reshape(1, F), mlp_W1,
            mlp_b1.reshape(1, F))

    out = pl.pallas_call(
        _gat_body,
        grid=(B,),
        in_specs=[pl.BlockSpec((1, N, F), lambda b: (b, 0, 0))]
        + [full(a) for a in args],
        out_specs=pl.BlockSpec((1, N, F), lambda b: (b, 0, 0)),
        out_shape=jax.ShapeDtypeStruct((B, N, F), jnp.float32),
    )(xin, *args)

    xo = jnp.transpose(out, (0, 2, 1))
    return xo[:, :, :S0], xo[:, :, S0:]


# speedup vs baseline: 1.4578x; 1.2620x over previous
import functools

import jax
import jax.numpy as jnp
from jax.experimental import pallas as pl

B = 4
F = 128
S0 = 256
S1 = 256
H = 4
N = S0 + S1

_dotg = functools.partial(
    jax.lax.dot_general,
    precision=jax.lax.Precision.DEFAULT,
    preferred_element_type=jnp.float32,
)


def _dot(a, b):
    return _dotg(a, b, (((1,), (0,)), ((), ())))


def _dot_t(a, b):
    return _dotg(a, b, (((1,), (1,)), ((), ())))


def _lrelu(x):
    return jnp.maximum(x, 0.2 * x)


def _gat_body(d0_ref, d1_ref,
              W0_ref, as0_ref, ad0_ref, b0_ref, mW0_ref, mb0_ref,
              W1_ref, as1_ref, ad1_ref, b1_ref, mW1_ref, mb1_ref,
              o0_ref, o1_ref):
    x0 = jnp.swapaxes(d0_ref[0], 0, 1)
    x1 = jnp.swapaxes(d1_ref[0], 0, 1)
    x = jnp.concatenate([x0, x1], axis=0)
    layers = (
        (W0_ref, as0_ref, ad0_ref, b0_ref, mW0_ref, mb0_ref, False),
        (W1_ref, as1_ref, ad1_ref, b1_ref, mW1_ref, mb1_ref, True),
    )
    for W_ref, as_ref, ad_ref, bias_ref, mW_ref, mb_ref, cross in layers:
        h = _dot(x, W_ref[...])
        msg_sets = []
        for s in (0, 1):
            dlo = s * S0
            slo = (1 - s) * S0 if cross else dlo
            acc = jnp.zeros((S0, F), jnp.float32)
            for hi in range(H):
                hs = h[slo:slo + S0, hi * F:(hi + 1) * F]
                hd = h[dlo:dlo + S0, hi * F:(hi + 1) * F]
                a_s = as_ref[hi:hi + 1, :]
                a_d = ad_ref[hi:hi + 1, :]
                row = _dot_t(a_s, hs)
                col = _dot_t(hd, a_d)
                sc = _lrelu(row + col)
                rmax = jnp.max(row, axis=1, keepdims=True)
                m = _lrelu(rmax + col)
                if cross:
                    s_self = _lrelu(_dot_t(hd, a_s) + col)
                    m = jnp.maximum(m, s_self)
                    e = jnp.exp(sc - m)
                    e_self = jnp.exp(s_self - m)
                    den = jnp.sum(e, axis=1, keepdims=True) + e_self + 1e-16
                    acc = acc + (_dot(e, hs) + e_self * hd) / den
                else:
                    e = jnp.exp(sc - m)
                    den = jnp.sum(e, axis=1, keepdims=True) + 1e-16
                    acc = acc + _dot(e, hs) / den
            msg_sets.append(acc)
        msg1 = jnp.concatenate(msg_sets, axis=0)
        msg1 = msg1 * (1.0 / H) + bias_ref[...]
        msg1 = jnp.maximum(msg1, 0.0)
        mW = mW_ref[...]
        msg2 = _dot(x, mW[:F, :]) + _dot(msg1, mW[F:, :]) + mb_ref[...]
        x = x + msg2
    xT = jnp.swapaxes(x, 0, 1)
    o0_ref[0] = xT[:, :S0]
    o1_ref[0] = xT[:, S0:]


@jax.jit
def kernel(desc0, desc1, W0, att_src0, att_dst0, b0, mlp_W0, mlp_b0,
           W1, att_src1, att_dst1, b1, mlp_W1, mlp_b1):
    full = lambda a: pl.BlockSpec(a.shape, lambda b: (0,) * a.ndim)
    args = (W0, att_src0, att_dst0, b0.reshape(1, F), mlp_W0,
            mlp_b0.reshape(1, F),
            W1, att_src1, att_dst1, b1.reshape(1, F), mlp_W1,
            mlp_b1.reshape(1, F))

    io_spec = pl.BlockSpec((1, F, S0), lambda b: (b, 0, 0))
    return pl.pallas_call(
        _gat_body,
        grid=(B,),
        in_specs=[io_spec, io_spec] + [full(a) for a in args],
        out_specs=(io_spec, io_spec),
        out_shape=(jax.ShapeDtypeStruct((B, F, S0), jnp.float32),
                   jax.ShapeDtypeStruct((B, F, S1), jnp.float32)),
    )(desc0, desc1, *args)


# speedup vs baseline: 1.4588x; 1.0007x over previous
import functools

import jax
import jax.numpy as jnp
from jax.experimental import pallas as pl
from jax.experimental.pallas import tpu as pltpu

B = 4
F = 128
S0 = 256
S1 = 256
H = 4
N = S0 + S1

_dotg = functools.partial(
    jax.lax.dot_general,
    precision=jax.lax.Precision.DEFAULT,
    preferred_element_type=jnp.float32,
)


def _dot(a, b):
    return _dotg(a, b, (((1,), (0,)), ((), ())))


def _dot_t(a, b):
    return _dotg(a, b, (((1,), (1,)), ((), ())))


def _lrelu(x):
    return jnp.maximum(x, 0.2 * x)


def _gat_body(d0_ref, d1_ref,
              W0_ref, as0_ref, ad0_ref, b0_ref, mW0_ref, mb0_ref,
              W1_ref, as1_ref, ad1_ref, b1_ref, mW1_ref, mb1_ref,
              o0_ref, o1_ref):
    x0 = jnp.swapaxes(d0_ref[0], 0, 1)
    x1 = jnp.swapaxes(d1_ref[0], 0, 1)
    x = jnp.concatenate([x0, x1], axis=0)
    layers = (
        (W0_ref, as0_ref, ad0_ref, b0_ref, mW0_ref, mb0_ref, False),
        (W1_ref, as1_ref, ad1_ref, b1_ref, mW1_ref, mb1_ref, True),
    )
    for W_ref, as_ref, ad_ref, bias_ref, mW_ref, mb_ref, cross in layers:
        h = _dot(x, W_ref[...])
        msg_sets = []
        for s in (0, 1):
            dlo = s * S0
            slo = (1 - s) * S0 if cross else dlo
            acc = jnp.zeros((S0, F), jnp.float32)
            for hi in range(H):
                hs = h[slo:slo + S0, hi * F:(hi + 1) * F]
                hd = h[dlo:dlo + S0, hi * F:(hi + 1) * F]
                a_s = as_ref[hi:hi + 1, :]
                a_d = ad_ref[hi:hi + 1, :]
                row = _dot_t(a_s, hs)
                col = _dot_t(hd, a_d)
                sc = _lrelu(row + col)
                rmax = jnp.max(row, axis=1, keepdims=True)
                m = _lrelu(rmax + col)
                if cross:
                    s_self = _lrelu(_dot_t(hd, a_s) + col)
                    m = jnp.maximum(m, s_self)
                    e = jnp.exp(sc - m)
                    e_self = jnp.exp(s_self - m)
                    den = jnp.sum(e, axis=1, keepdims=True) + e_self + 1e-16
                    acc = acc + (_dot(e, hs) + e_self * hd) / den
                else:
                    e = jnp.exp(sc - m)
                    den = jnp.sum(e, axis=1, keepdims=True) + 1e-16
                    acc = acc + _dot(e, hs) / den
            msg_sets.append(acc)
        msg1 = jnp.concatenate(msg_sets, axis=0)
        msg1 = msg1 * (1.0 / H) + bias_ref[...]
        msg1 = jnp.maximum(msg1, 0.0)
        mW = mW_ref[...]
        msg2 = _dot(x, mW[:F, :]) + _dot(msg1, mW[F:, :]) + mb_ref[...]
        x = x + msg2
    xT = jnp.swapaxes(x, 0, 1)
    o0_ref[0] = xT[:, :S0]
    o1_ref[0] = xT[:, S0:]


@jax.jit
def kernel(desc0, desc1, W0, att_src0, att_dst0, b0, mlp_W0, mlp_b0,
           W1, att_src1, att_dst1, b1, mlp_W1, mlp_b1):
    full = lambda a: pl.BlockSpec(a.shape, lambda b: (0,) * a.ndim)
    args = (W0, att_src0, att_dst0, b0.reshape(1, F), mlp_W0,
            mlp_b0.reshape(1, F),
            W1, att_src1, att_dst1, b1.reshape(1, F), mlp_W1,
            mlp_b1.reshape(1, F))

    io_spec = pl.BlockSpec((1, F, S0), lambda b: (b, 0, 0))
    return pl.pallas_call(
        _gat_body,
        grid=(B,),
        in_specs=[io_spec, io_spec] + [full(a) for a in args],
        out_specs=(io_spec, io_spec),
        out_shape=(jax.ShapeDtypeStruct((B, F, S0), jnp.float32),
                   jax.ShapeDtypeStruct((B, F, S1), jnp.float32)),
        compiler_params=pltpu.CompilerParams(
            dimension_semantics=("parallel",)),
    )(desc0, desc1, *args)
